# baseline (speedup 1.0000x reference)
import math

import jax
import jax.numpy as jnp
from jax.experimental import pallas as pl
from jax.experimental.pallas import tpu as pltpu

D_MODEL = 2048
HIDDEN = 256
NUM_EXPERTS = 64
TOP_K = 2
N_TOK = 16384

TILE = 2048

_INV_SQRT2 = 1.0 / math.sqrt(2.0)


def _fused_gating_kernel(x_ref, w1_hbm, b1_hbm, w2_hbm, b2_hbm,
                         w_out_ref, i_out_ref,
                         w1_ref, b1_ref, w2_ref, b2_ref, sems):
    @pl.when(pl.program_id(0) == 0)
    def _():
        pltpu.make_async_copy(w1_hbm, w1_ref, sems.at[0]).start()
        pltpu.make_async_copy(b1_hbm, b1_ref, sems.at[1]).start()
        pltpu.make_async_copy(w2_hbm, w2_ref, sems.at[2]).start()
        pltpu.make_async_copy(b2_hbm, b2_ref, sems.at[3]).start()
        pltpu.make_async_copy(w1_hbm, w1_ref, sems.at[0]).wait()
        pltpu.make_async_copy(b1_hbm, b1_ref, sems.at[1]).wait()
        pltpu.make_async_copy(w2_hbm, w2_ref, sems.at[2]).wait()
        pltpu.make_async_copy(b2_hbm, b2_ref, sems.at[3]).wait()

    h = jnp.dot(x_ref[...], w1_ref[...], preferred_element_type=jnp.float32)
    h = h + b1_ref[...]
    h = 0.5 * h * (1.0 + jax.lax.erf(h * _INV_SQRT2))
    logits = jnp.dot(h, w2_ref[...], preferred_element_type=jnp.float32)
    logits = logits + b2_ref[...]

    col = jax.lax.broadcasted_iota(jnp.int32, logits.shape, 1)
    revcol = (NUM_EXPERTS - 1 - col).astype(jnp.float32)
    m1 = jnp.max(logits, axis=1, keepdims=True)
    r1 = jnp.max(jnp.where(logits == m1, revcol, -1.0), axis=1,
                 keepdims=True)
    i1 = (NUM_EXPERTS - 1) - r1.astype(jnp.int32)
    masked = jnp.where(col == i1, -jnp.inf, logits)
    m2 = jnp.max(masked, axis=1, keepdims=True)
    r2 = jnp.max(jnp.where(masked == m2, revcol, -1.0), axis=1,
                 keepdims=True)
    i2 = (NUM_EXPERTS - 1) - r2.astype(jnp.int32)

    e2 = jnp.exp(m2 - m1)
    denom = 1.0 + e2
    w_out_ref[...] = jnp.concatenate([1.0 / denom, e2 / denom], axis=1)
    i_out_ref[...] = jnp.concatenate([i1, i2], axis=1)


@jax.jit
def kernel(x, W1, b1, W2, b2):
    b1r = b1.reshape(1, HIDDEN)
    b2r = b2.reshape(1, NUM_EXPERTS)
    grid = (N_TOK // TILE,)
    weights, topk_i = pl.pallas_call(
        _fused_gating_kernel,
        grid=grid,
        in_specs=[
            pl.BlockSpec((TILE, D_MODEL), lambda i: (i, 0)),
            pl.BlockSpec(memory_space=pl.ANY),
            pl.BlockSpec(memory_space=pl.ANY),
            pl.BlockSpec(memory_space=pl.ANY),
            pl.BlockSpec(memory_space=pl.ANY),
        ],
        out_specs=[
            pl.BlockSpec((TILE, TOP_K), lambda i: (i, 0)),
            pl.BlockSpec((TILE, TOP_K), lambda i: (i, 0)),
        ],
        out_shape=[
            jax.ShapeDtypeStruct((N_TOK, TOP_K), jnp.float32),
            jax.ShapeDtypeStruct((N_TOK, TOP_K), jnp.int32),
        ],
        scratch_shapes=[
            pltpu.VMEM((D_MODEL, HIDDEN), jnp.float32),
            pltpu.VMEM((1, HIDDEN), jnp.float32),
            pltpu.VMEM((HIDDEN, NUM_EXPERTS), jnp.float32),
            pltpu.VMEM((1, NUM_EXPERTS), jnp.float32),
            pltpu.SemaphoreType.DMA((4,)),
        ],
        compiler_params=pltpu.CompilerParams(
            dimension_semantics=("arbitrary",),
        ),
    )(x, W1, b1r, W2, b2r)
    return (weights, topk_i)
